# baseline (speedup 1.0000x reference)
import functools

import jax
import jax.numpy as jnp
from jax import lax
from jax.experimental import pallas as pl
from jax.experimental.pallas import tpu as pltpu
from jax.experimental.pallas import tpu_sc as plsc

CHUNK = 128
NBUF = 8
AHEAD = 4


@functools.cache
def _build(B, V, D):
    info = plsc.get_sparse_core_info()
    NC, NS = info.num_cores, info.num_subcores
    NW = NC * NS
    assert B % (NW * CHUNK) == 0
    b_per_w = B // NW
    n_chunks = b_per_w // CHUNK
    assert n_chunks % NBUF == 0 and n_chunks >= 2 * NBUF

    mesh = plsc.VectorSubcoreMesh(core_axis_name="c", subcore_axis_name="s")

    @functools.partial(
        pl.kernel,
        mesh=mesh,
        compiler_params=pltpu.CompilerParams(use_tc_tiling_on_sc=False),
        out_type=jax.ShapeDtypeStruct((B, D), jnp.float32),
        scratch_types=[
            pltpu.VMEM((n_chunks, CHUNK), jnp.int32),
            pltpu.VMEM((NBUF, CHUNK, D), jnp.float32),
            pltpu.SemaphoreType.DMA((NBUF,)),
            pltpu.SemaphoreType.DMA((NBUF,)),
        ],
    )
    def emb_kernel(idx_hbm, table_hbm, out_hbm, idx_v, rows_v, gsem, ssem):
        wid = lax.axis_index("s") * NC + lax.axis_index("c")
        pltpu.sync_copy(idx_hbm.at[wid], idx_v)
        base = wid * b_per_w

        def fire_gather(j, b):
            pltpu.async_copy(table_hbm.at[idx_v.at[j]], rows_v.at[b],
                             gsem.at[b])

        def wait_gather(j, b):
            pltpu.make_async_copy(table_hbm.at[idx_v.at[j]], rows_v.at[b],
                                  gsem.at[b]).wait()

        def fire_store(j, b):
            pltpu.async_copy(rows_v.at[b],
                             out_hbm.at[pl.ds(base + j * CHUNK, CHUNK)],
                             ssem.at[b])

        def wait_store(j, b):
            pltpu.make_async_copy(rows_v.at[b],
                                  out_hbm.at[pl.ds(base + j * CHUNK, CHUNK)],
                                  ssem.at[b]).wait()

        for b in range(AHEAD):
            fire_gather(b, b)

        def super_round(t, carry):
            for b in range(NBUF):
                j = t * NBUF + b
                bg = (b + AHEAD) % NBUF
                wait_store(j + AHEAD - NBUF, bg)
                fire_gather(j + AHEAD, bg)
                wait_gather(j, b)
                fire_store(j, b)
            return carry

        for b in range(NBUF):
            bg = (b + AHEAD) % NBUF
            if b < NBUF - AHEAD:
                fire_gather(b + AHEAD, bg)
            else:
                wait_store(b + AHEAD - NBUF, bg)
                fire_gather(b + AHEAD, bg)
            wait_gather(b, b)
            fire_store(b, b)

        lax.fori_loop(1, n_chunks // NBUF - 1, super_round, 0, unroll=False)

        t_last = n_chunks // NBUF - 1
        for b in range(NBUF):
            j = t_last * NBUF + b
            bg = (b + AHEAD) % NBUF
            if b < NBUF - AHEAD:
                wait_store(j + AHEAD - NBUF, bg)
                fire_gather(j + AHEAD, bg)
            wait_gather(j, b)
            fire_store(j, b)

        for i in range(NBUF):
            wait_store(n_chunks - NBUF + i, i)

    return emb_kernel


def kernel(x, table):
    B0, S = x.shape
    V, D = table.shape
    B = B0 * S
    info = plsc.get_sparse_core_info()
    NW = info.num_cores * info.num_subcores
    idx = x.reshape(NW, (B // NW) // CHUNK, CHUNK).astype(jnp.int32)
    out = _build(B, V, D)(idx, table)
    return out.reshape(B0, S, D)


# speedup vs baseline: 1.0002x; 1.0002x over previous
import functools

import jax
import jax.numpy as jnp
from jax import lax
from jax.experimental import pallas as pl
from jax.experimental.pallas import tpu as pltpu
from jax.experimental.pallas import tpu_sc as plsc

NBUF = 8
AHEAD = 4


@functools.cache
def _build(B0, S, V, D):
    info = plsc.get_sparse_core_info()
    NC, NS = info.num_cores, info.num_subcores
    NW = NC * NS
    assert B0 % NW == 0
    rows_per_w = B0 // NW
    H0 = min(128, (S // 2 + 7) // 8 * 8)
    H1 = S - H0
    assert 0 < H1 <= 128 and H0 % 8 == 0 and H1 % 8 == 0
    n_chunks = 2 * rows_per_w
    assert n_chunks % NBUF == 0 and n_chunks >= 2 * NBUF

    mesh = plsc.VectorSubcoreMesh(core_axis_name="c", subcore_axis_name="s")

    @functools.partial(
        pl.kernel,
        mesh=mesh,
        compiler_params=pltpu.CompilerParams(use_tc_tiling_on_sc=False),
        out_type=jax.ShapeDtypeStruct((B0, S, D), jnp.float32),
        scratch_types=[
            pltpu.VMEM((rows_per_w, S), jnp.int32),
            pltpu.VMEM((NBUF, H0, D), jnp.float32),
            pltpu.SemaphoreType.DMA((NBUF,)),
            pltpu.SemaphoreType.DMA((NBUF,)),
        ],
    )
    def emb_kernel(x_hbm, table_hbm, out_hbm, idx_v, rows_v, gsem, ssem):
        wid = lax.axis_index("s") * NC + lax.axis_index("c")
        base_b = wid * rows_per_w
        pltpu.sync_copy(x_hbm.at[pl.ds(base_b, rows_per_w)], idx_v)

        def _span(c, par):
            off, ln = (0, H0) if par == 0 else (H0, H1)
            return c // 2, off, ln

        def fire_gather(c, b, par):
            r, off, ln = _span(c, par)
            pltpu.async_copy(
                table_hbm.at[idx_v.at[r, pl.ds(off, ln)]],
                rows_v.at[b, pl.ds(0, ln)], gsem.at[b])

        def wait_gather(c, b, par):
            r, off, ln = _span(c, par)
            pltpu.make_async_copy(
                table_hbm.at[idx_v.at[r, pl.ds(off, ln)]],
                rows_v.at[b, pl.ds(0, ln)], gsem.at[b]).wait()

        def fire_store(c, b, par):
            r, off, ln = _span(c, par)
            pltpu.async_copy(
                rows_v.at[b, pl.ds(0, ln)],
                out_hbm.at[base_b + r, pl.ds(off, ln)], ssem.at[b])

        def wait_store(c, b, par):
            r, off, ln = _span(c, par)
            pltpu.make_async_copy(
                rows_v.at[b, pl.ds(0, ln)],
                out_hbm.at[base_b + r, pl.ds(off, ln)],
                ssem.at[b]).wait()

        for b in range(AHEAD):
            fire_gather(b, b, b % 2)

        def super_round(t, carry):
            for b in range(NBUF):
                c = t * NBUF + b
                bg = (b + AHEAD) % NBUF
                wait_store(c + AHEAD - NBUF, bg, b % 2)
                fire_gather(c + AHEAD, bg, b % 2)
                wait_gather(c, b, b % 2)
                fire_store(c, b, b % 2)
            return carry

        for b in range(NBUF):
            bg = (b + AHEAD) % NBUF
            if b >= NBUF - AHEAD:
                wait_store(b + AHEAD - NBUF, bg, b % 2)
            fire_gather(b + AHEAD, bg, b % 2)
            wait_gather(b, b, b % 2)
            fire_store(b, b, b % 2)

        lax.fori_loop(1, n_chunks // NBUF - 1, super_round, 0)

        t_last = n_chunks // NBUF - 1
        for b in range(NBUF):
            c = t_last * NBUF + b
            bg = (b + AHEAD) % NBUF
            if b < NBUF - AHEAD:
                wait_store(c + AHEAD - NBUF, bg, b % 2)
                fire_gather(c + AHEAD, bg, b % 2)
            wait_gather(c, b, b % 2)
            fire_store(c, b, b % 2)

        for i in range(NBUF):
            wait_store(n_chunks - NBUF + i, i, i % 2)

    return emb_kernel


def kernel(x, table):
    B0, S = x.shape
    V, D = table.shape
    return _build(B0, S, V, D)(x.astype(jnp.int32), table)
